# baseline (speedup 1.0000x reference)
import functools

import jax
import jax.numpy as jnp
from jax import lax
from jax.experimental import pallas as pl
from jax.experimental.pallas import tpu as pltpu
from jax.experimental.pallas import tpu_sc as plsc

_N = 10000
_E = 320000
_ND = 128
_ED = 16
_H = 128

_NC = 2
_NS = 16
_NW = _NC * _NS
_LANES = 16

_GCH = 128
_NCHUNK = _E // _GCH
_NP = 10240


def _gelu(x):
    c = 0.7978845608028654
    return 0.5 * x * (1.0 + jnp.tanh(c * (x + 0.044715 * x * x * x)))


def _proj_body(x_ref, wa_ref, wb_ref, ps_ref, pr_ref):
    x = x_ref[...]
    ps_ref[...] = jnp.dot(x, wa_ref[...], preferred_element_type=jnp.float32).astype(jnp.bfloat16)
    pr_ref[...] = jnp.dot(x, wb_ref[...], preferred_element_type=jnp.float32).astype(jnp.bfloat16)


def _proj_tables(nodes, wa, wb):
    n, nd = nodes.shape
    h = wa.shape[1]
    bn = 1000
    return pl.pallas_call(
        _proj_body,
        grid=(n // bn,),
        in_specs=[
            pl.BlockSpec((bn, nd), lambda i: (i, 0)),
            pl.BlockSpec((nd, h), lambda i: (0, 0)),
            pl.BlockSpec((nd, h), lambda i: (0, 0)),
        ],
        out_specs=[
            pl.BlockSpec((bn, h), lambda i: (i, 0)),
            pl.BlockSpec((bn, h), lambda i: (i, 0)),
        ],
        out_shape=[
            jax.ShapeDtypeStruct((n, h), jnp.bfloat16),
            jax.ShapeDtypeStruct((n, h), jnp.bfloat16),
        ],
    )(nodes, wa, wb)


def _gather_body(ps_hbm, pr_hbm, snd_hbm, rcv_hbm, out_hbm,
                 sidx0, ridx0, sidx1, ridx1,
                 bufs0, bufr0, bufs1, bufr1, obuf0, obuf1,
                 gsem0, gsem1, wsem0, wsem1):
    cid = lax.axis_index("c")
    sid = lax.axis_index("s")
    wid = sid * _NC + cid
    nfull = _NCHUNK // _NW
    extra = _NCHUNK - nfull * _NW
    nch = nfull + jnp.where(wid < extra, 1, 0)
    nmax = nfull + (1 if extra else 0)
    ngrp = (nmax + 1) // 2

    slots = ((sidx0, ridx0, bufs0, bufr0, obuf0, gsem0, wsem0),
             (sidx1, ridx1, bufs1, bufr1, obuf1, gsem1, wsem1))

    def load_idx(i, si, ri):
        ch = wid + i * _NW
        pltpu.sync_copy(snd_hbm.at[ch], si)
        pltpu.sync_copy(rcv_hbm.at[ch], ri)

    def start_gather(si, ri, bs, br, sem):
        pltpu.async_copy(ps_hbm.at[si], bs, sem)
        pltpu.async_copy(pr_hbm.at[ri], br, sem)

    def drain_gather(si, ri, bs, br, sem):
        pltpu.make_async_copy(ps_hbm.at[si], bs, sem).wait()
        pltpu.make_async_copy(pr_hbm.at[ri], br, sem).wait()

    def process(i, p):
        si, ri, bs, br, ob, gsem, wsem = slots[p]
        drain_gather(si, ri, bs, br, gsem)

        @pl.when(i >= 2)
        def _():
            pltpu.make_async_copy(ob, out_hbm.at[pl.ds(0, _GCH)], wsem).wait()

        def add_row(r, c2):
            for c in range(_H // (2 * _LANES)):
                sl = pl.ds(c * 2 * _LANES, 2 * _LANES)
                ob[r, sl] = bs[r, sl] + br[r, sl]
            return c2

        lax.fori_loop(0, _GCH, add_row, 0)
        ch = wid + i * _NW
        pltpu.async_copy(ob, out_hbm.at[pl.ds(ch * _GCH, _GCH)], wsem)

    load_idx(0, sidx0, ridx0)
    start_gather(sidx0, ridx0, bufs0, bufr0, gsem0)

    def group(g, carry):
        a = 2 * g
        b = 2 * g + 1
        a2 = 2 * g + 2

        @pl.when(b < nch)
        def _():
            load_idx(b, sidx1, ridx1)
            start_gather(sidx1, ridx1, bufs1, bufr1, gsem1)

        @pl.when(a < nch)
        def _():
            process(a, 0)

        @pl.when(a2 < nch)
        def _():
            load_idx(a2, sidx0, ridx0)
            start_gather(sidx0, ridx0, bufs0, bufr0, gsem0)

        @pl.when(b < nch)
        def _():
            process(b, 1)

        return carry

    lax.fori_loop(0, ngrp, group, 0)
    pltpu.make_async_copy(obuf0, out_hbm.at[pl.ds(0, _GCH)], wsem0).wait()
    pltpu.make_async_copy(obuf1, out_hbm.at[pl.ds(0, _GCH)], wsem1).wait()


def _gather_sum(ps, pr, snd2d, rcv2d):
    mesh = plsc.VectorSubcoreMesh(core_axis_name="c", subcore_axis_name="s")
    fn = functools.partial(
        pl.kernel,
        mesh=mesh,
        out_type=jax.ShapeDtypeStruct((_E, _H), jnp.bfloat16),
        compiler_params=pltpu.CompilerParams(use_tc_tiling_on_sc=False),
        scratch_types=[
            pltpu.VMEM((_GCH,), jnp.int32),
            pltpu.VMEM((_GCH,), jnp.int32),
            pltpu.VMEM((_GCH,), jnp.int32),
            pltpu.VMEM((_GCH,), jnp.int32),
            pltpu.VMEM((_GCH, _H), jnp.bfloat16),
            pltpu.VMEM((_GCH, _H), jnp.bfloat16),
            pltpu.VMEM((_GCH, _H), jnp.bfloat16),
            pltpu.VMEM((_GCH, _H), jnp.bfloat16),
            pltpu.VMEM((_GCH, _H), jnp.bfloat16),
            pltpu.VMEM((_GCH, _H), jnp.bfloat16),
            pltpu.SemaphoreType.DMA,
            pltpu.SemaphoreType.DMA,
            pltpu.SemaphoreType.DMA,
            pltpu.SemaphoreType.DMA,
        ],
    )(_gather_body)
    return fn(ps, pr, snd2d, rcv2d)


def _edge_body(g_ref, e_ref, w1c_ref, b1_ref, w2_ref, b2_ref, ne_ref, oe_ref):
    e = e_ref[...]
    x = g_ref[...].astype(jnp.float32) + jnp.dot(e, w1c_ref[...], preferred_element_type=jnp.float32)
    x = x + b1_ref[...]
    h = _gelu(x)
    ne = jnp.dot(h, w2_ref[...], preferred_element_type=jnp.float32) + b2_ref[...]
    ne_ref[...] = ne
    oe_ref[...] = e + ne


def _edge_mlp(g, edges, w1c, b1, w2, b2):
    e, ed = edges.shape
    h = g.shape[1]
    be = 2000
    return pl.pallas_call(
        _edge_body,
        grid=(e // be,),
        in_specs=[
            pl.BlockSpec((be, h), lambda i: (i, 0)),
            pl.BlockSpec((be, ed), lambda i: (i, 0)),
            pl.BlockSpec((ed, h), lambda i: (0, 0)),
            pl.BlockSpec((1, h), lambda i: (0, 0)),
            pl.BlockSpec((h, ed), lambda i: (0, 0)),
            pl.BlockSpec((1, ed), lambda i: (0, 0)),
        ],
        out_specs=[
            pl.BlockSpec((be, ed), lambda i: (i, 0)),
            pl.BlockSpec((be, ed), lambda i: (i, 0)),
        ],
        out_shape=[
            jax.ShapeDtypeStruct((e, ed), jnp.float32),
            jax.ShapeDtypeStruct((e, ed), jnp.float32),
        ],
    )(g, edges, w1c, b1.reshape(1, h), w2, b2.reshape(1, ed))


_NSLOT = 4


def _scatter_body(ne_hbm, rcv_hbm, zeros_hbm, out_hbm,
                  ridx0, ridx1, ridx2, ridx3,
                  vals0, vals1, vals2, vals3,
                  st0, st1, st2, st3, sc0, sc1, sc2, sc3, agg_sh):
    cid = lax.axis_index("c")
    sid = lax.axis_index("s")
    wid = sid * _NC + cid
    nfull = _NCHUNK // _NW
    extra = _NCHUNK - nfull * _NW
    nch = nfull + jnp.where(wid < extra, 1, 0)
    nmax = nfull + (1 if extra else 0)
    ngrp = (nmax + _NSLOT - 1) // _NSLOT

    slots = ((ridx0, vals0, st0, sc0), (ridx1, vals1, st1, sc1),
             (ridx2, vals2, st2, sc2), (ridx3, vals3, st3, sc3))

    @pl.when(sid == 0)
    def _zero():
        pltpu.sync_copy(zeros_hbm, agg_sh)

    plsc.subcore_barrier()

    def stage(i, p):
        ri, va, st, _ = slots[p]
        ch = wid + i * _NW
        pltpu.async_copy(rcv_hbm.at[ch], ri, st)
        pltpu.async_copy(ne_hbm.at[pl.ds(ch * _GCH, _GCH)], va, st)

    def drain_stage(p):
        ri, va, st, _ = slots[p]
        pltpu.make_async_copy(rcv_hbm.at[0], ri, st).wait()
        pltpu.make_async_copy(ne_hbm.at[pl.ds(0, _GCH)], va, st).wait()

    def drain_scatter(p):
        ri, va, _, sc = slots[p]
        pltpu.make_async_copy(va, agg_sh.at[ri], sc).wait()

    for p in range(_NSLOT):
        @pl.when(p < nch)
        def _(p=p):
            stage(p, p)

    def group(g, carry):
        for p in range(_NSLOT):
            i = _NSLOT * g + p

            @pl.when(i < nch)
            def _(i=i, p=p):
                ri, va, _, sc = slots[p]
                drain_stage(p)
                pltpu.async_copy(va, agg_sh.at[ri], sc, add=True)

        for p in range(_NSLOT):
            j = _NSLOT * (g + 1) + p

            @pl.when(j < nch)
            def _(j=j, p=p):
                drain_scatter(p)
                stage(j, p)

        return carry

    lax.fori_loop(0, ngrp, group, 0)
    for p in range(_NSLOT):
        drain_scatter(p)
    plsc.subcore_barrier()

    @pl.when(sid == 0)
    def _writeback():
        pltpu.sync_copy(agg_sh, out_hbm.at[cid])


def _scatter_add(new_edges, rcv2d):
    mesh = plsc.VectorSubcoreMesh(core_axis_name="c", subcore_axis_name="s")
    zeros = jnp.zeros((_NP, _ED), jnp.float32)
    fn = functools.partial(
        pl.kernel,
        mesh=mesh,
        out_type=jax.ShapeDtypeStruct((_NC, _NP, _ED), jnp.float32),
        compiler_params=pltpu.CompilerParams(use_tc_tiling_on_sc=False),
        scratch_types=[
            pltpu.VMEM((_GCH,), jnp.int32),
            pltpu.VMEM((_GCH,), jnp.int32),
            pltpu.VMEM((_GCH,), jnp.int32),
            pltpu.VMEM((_GCH,), jnp.int32),
            pltpu.VMEM((_GCH, _ED), jnp.float32),
            pltpu.VMEM((_GCH, _ED), jnp.float32),
            pltpu.VMEM((_GCH, _ED), jnp.float32),
            pltpu.VMEM((_GCH, _ED), jnp.float32),
            pltpu.SemaphoreType.DMA,
            pltpu.SemaphoreType.DMA,
            pltpu.SemaphoreType.DMA,
            pltpu.SemaphoreType.DMA,
            pltpu.SemaphoreType.DMA,
            pltpu.SemaphoreType.DMA,
            pltpu.SemaphoreType.DMA,
            pltpu.SemaphoreType.DMA,
            pltpu.VMEM_SHARED((_NP, _ED), jnp.float32),
        ],
    )(_scatter_body)
    return fn(new_edges, rcv2d, zeros)


def _node_body(x_ref, agg_ref, w1a_ref, w1b_ref, b1_ref, w2_ref, b2_ref, out_ref):
    x = x_ref[...]
    a = agg_ref[0] + agg_ref[1]
    t = jnp.dot(x, w1a_ref[...], preferred_element_type=jnp.float32)
    t = t + jnp.dot(a, w1b_ref[...], preferred_element_type=jnp.float32)
    t = t + b1_ref[...]
    h = _gelu(t)
    out_ref[...] = x + jnp.dot(h, w2_ref[...], preferred_element_type=jnp.float32) + b2_ref[...]


def _node_mlp(nodes, agg2, w1a, w1b, b1, w2, b2):
    n, nd = nodes.shape
    ed = agg2.shape[2]
    h = w1a.shape[1]
    bn = 1000
    return pl.pallas_call(
        _node_body,
        grid=(n // bn,),
        in_specs=[
            pl.BlockSpec((bn, nd), lambda i: (i, 0)),
            pl.BlockSpec((_NC, bn, ed), lambda i: (0, i, 0)),
            pl.BlockSpec((nd, h), lambda i: (0, 0)),
            pl.BlockSpec((ed, h), lambda i: (0, 0)),
            pl.BlockSpec((1, h), lambda i: (0, 0)),
            pl.BlockSpec((h, nd), lambda i: (0, 0)),
            pl.BlockSpec((1, nd), lambda i: (0, 0)),
        ],
        out_specs=[pl.BlockSpec((bn, nd), lambda i: (i, 0))],
        out_shape=[jax.ShapeDtypeStruct((n, nd), jnp.float32)],
    )(nodes, agg2, w1a, w1b, b1.reshape(1, h), w2, b2.reshape(1, nd))[0]


def kernel(nodes, edges, receivers, senders,
           edge_W1, edge_b1, edge_W2, edge_b2,
           node_W1, node_b1, node_W2, node_b2):
    n, nd = nodes.shape
    e, ed = edges.shape
    assert (n, e, nd, ed) == (_N, _E, _ND, _ED)

    ew1a = edge_W1[:nd]
    ew1b = edge_W1[nd:2 * nd]
    ew1c = edge_W1[2 * nd:]
    nw1a = node_W1[:nd]
    nw1b = node_W1[nd:]

    snd2d = senders.reshape(_NCHUNK, _GCH)
    rcv2d = receivers.reshape(_NCHUNK, _GCH)

    ps, pr = _proj_tables(nodes, ew1a, ew1b)
    g = _gather_sum(ps, pr, snd2d, rcv2d)
    new_edges, out_edges = _edge_mlp(g, edges, ew1c, edge_b1, edge_W2, edge_b2)
    aggp = _scatter_add(new_edges, rcv2d)
    agg2 = aggp[:, :_N]
    out_nodes = _node_mlp(nodes, agg2, nw1a, nw1b, node_b1, node_W2, node_b2)
    return out_nodes, out_edges


# speedup vs baseline: 1.3934x; 1.3934x over previous
import functools

import jax
import jax.numpy as jnp
from jax import lax
from jax.experimental import pallas as pl
from jax.experimental.pallas import tpu as pltpu
from jax.experimental.pallas import tpu_sc as plsc

_N = 10000
_E = 320000
_ND = 128
_ED = 16
_H = 128

_NC = 2
_NS = 16
_NW = _NC * _NS
_LANES = 16

_GCH = 128
_NCHUNK = _E // _GCH
_NP = 10240


def _gelu(x):
    c = 0.7978845608028654
    return 0.5 * x * (1.0 + jnp.tanh(c * (x + 0.044715 * x * x * x)))


def _proj_body(x_ref, wa_ref, wb_ref, ps_ref, pr_ref):
    x = x_ref[...]
    ps_ref[...] = jnp.dot(x, wa_ref[...], preferred_element_type=jnp.float32)
    pr_ref[...] = jnp.dot(x, wb_ref[...], preferred_element_type=jnp.float32)


def _proj_tables(nodes, wa, wb):
    n, nd = nodes.shape
    h = wa.shape[1]
    bn = 1000
    return pl.pallas_call(
        _proj_body,
        grid=(n // bn,),
        in_specs=[
            pl.BlockSpec((bn, nd), lambda i: (i, 0)),
            pl.BlockSpec((nd, h), lambda i: (0, 0)),
            pl.BlockSpec((nd, h), lambda i: (0, 0)),
        ],
        out_specs=[
            pl.BlockSpec((bn, h), lambda i: (i, 0)),
            pl.BlockSpec((bn, h), lambda i: (i, 0)),
        ],
        out_shape=[
            jax.ShapeDtypeStruct((n, h), jnp.float32),
            jax.ShapeDtypeStruct((n, h), jnp.float32),
        ],
    )(nodes, wa, wb)


def _gather_body(ps_hbm, pr_hbm, snd_hbm, rcv_hbm, out_hbm,
                 sidx0, ridx0, sidx1, ridx1,
                 bufs0, bufr0, bufs1, bufr1, obuf0, obuf1,
                 gsem0, gsem1, wsem0, wsem1):
    cid = lax.axis_index("c")
    sid = lax.axis_index("s")
    wid = sid * _NC + cid
    nfull = _NCHUNK // _NW
    extra = _NCHUNK - nfull * _NW
    nch = nfull + jnp.where(wid < extra, 1, 0)
    nmax = nfull + (1 if extra else 0)
    ngrp = (nmax + 1) // 2

    slots = ((sidx0, ridx0, bufs0, bufr0, obuf0, gsem0, wsem0),
             (sidx1, ridx1, bufs1, bufr1, obuf1, gsem1, wsem1))

    def load_idx(i, si, ri):
        ch = wid + i * _NW
        pltpu.sync_copy(snd_hbm.at[ch], si)
        pltpu.sync_copy(rcv_hbm.at[ch], ri)

    def start_gather(si, ri, bs, br, sem):
        pltpu.async_copy(ps_hbm.at[si], bs, sem)
        pltpu.async_copy(pr_hbm.at[ri], br, sem)

    def drain_gather(si, ri, bs, br, sem):
        pltpu.make_async_copy(ps_hbm.at[si], bs, sem).wait()
        pltpu.make_async_copy(pr_hbm.at[ri], br, sem).wait()

    def process(i, p):
        si, ri, bs, br, ob, gsem, wsem = slots[p]
        drain_gather(si, ri, bs, br, gsem)

        @pl.when(i >= 2)
        def _():
            pltpu.make_async_copy(ob, out_hbm.at[pl.ds(0, _GCH)], wsem).wait()

        def add_row(r, c2):
            for c in range(_H // _LANES):
                sl = pl.ds(c * _LANES, _LANES)
                ob[r, sl] = bs[r, sl] + br[r, sl]
            return c2

        lax.fori_loop(0, _GCH, add_row, 0)
        ch = wid + i * _NW
        pltpu.async_copy(ob, out_hbm.at[pl.ds(ch * _GCH, _GCH)], wsem)

    load_idx(0, sidx0, ridx0)
    start_gather(sidx0, ridx0, bufs0, bufr0, gsem0)

    def group(g, carry):
        a = 2 * g
        b = 2 * g + 1
        a2 = 2 * g + 2

        @pl.when(b < nch)
        def _():
            load_idx(b, sidx1, ridx1)
            start_gather(sidx1, ridx1, bufs1, bufr1, gsem1)

        @pl.when(a < nch)
        def _():
            process(a, 0)

        @pl.when(a2 < nch)
        def _():
            load_idx(a2, sidx0, ridx0)
            start_gather(sidx0, ridx0, bufs0, bufr0, gsem0)

        @pl.when(b < nch)
        def _():
            process(b, 1)

        return carry

    lax.fori_loop(0, ngrp, group, 0)
    pltpu.make_async_copy(obuf0, out_hbm.at[pl.ds(0, _GCH)], wsem0).wait()
    pltpu.make_async_copy(obuf1, out_hbm.at[pl.ds(0, _GCH)], wsem1).wait()


def _gather_sum(ps, pr, snd2d, rcv2d):
    mesh = plsc.VectorSubcoreMesh(core_axis_name="c", subcore_axis_name="s")
    fn = functools.partial(
        pl.kernel,
        mesh=mesh,
        out_type=jax.ShapeDtypeStruct((_E, _H), jnp.float32),
        scratch_types=[
            pltpu.VMEM((_GCH,), jnp.int32),
            pltpu.VMEM((_GCH,), jnp.int32),
            pltpu.VMEM((_GCH,), jnp.int32),
            pltpu.VMEM((_GCH,), jnp.int32),
            pltpu.VMEM((_GCH, _H), jnp.float32),
            pltpu.VMEM((_GCH, _H), jnp.float32),
            pltpu.VMEM((_GCH, _H), jnp.float32),
            pltpu.VMEM((_GCH, _H), jnp.float32),
            pltpu.VMEM((_GCH, _H), jnp.float32),
            pltpu.VMEM((_GCH, _H), jnp.float32),
            pltpu.SemaphoreType.DMA,
            pltpu.SemaphoreType.DMA,
            pltpu.SemaphoreType.DMA,
            pltpu.SemaphoreType.DMA,
        ],
    )(_gather_body)
    return fn(ps, pr, snd2d, rcv2d)


def _edge_body(g_ref, e_ref, w1c_ref, b1_ref, w2_ref, b2_ref, nep_ref, oe_ref):
    e = e_ref[...]
    x = g_ref[...] + jnp.dot(e, w1c_ref[...], preferred_element_type=jnp.float32)
    x = x + b1_ref[...]
    h = _gelu(x)
    ne = jnp.dot(h, w2_ref[...], preferred_element_type=jnp.float32) + b2_ref[...]
    nep_ref[...] = ne
    oe_ref[...] = e + ne


def _edge_mlp(g, edges, w1c, b1, w2, b2):
    e, ed = edges.shape
    h = g.shape[1]
    be = 2560
    pack = h // ed
    return pl.pallas_call(
        _edge_body,
        grid=(e // be,),
        in_specs=[
            pl.BlockSpec((be, h), lambda i: (i, 0)),
            pl.BlockSpec((be, ed), lambda i: (i, 0)),
            pl.BlockSpec((ed, h), lambda i: (0, 0)),
            pl.BlockSpec((1, h), lambda i: (0, 0)),
            pl.BlockSpec((h, ed), lambda i: (0, 0)),
            pl.BlockSpec((1, ed), lambda i: (0, 0)),
        ],
        out_specs=[
            pl.BlockSpec((be, ed), lambda i: (i, 0)),
            pl.BlockSpec((be, ed), lambda i: (i, 0)),
        ],
        out_shape=[
            jax.ShapeDtypeStruct((e, ed), jnp.float32),
            jax.ShapeDtypeStruct((e, ed), jnp.float32),
        ],
    )(g, edges, w1c, b1.reshape(1, h), w2, b2.reshape(1, ed))


_NSLOT = 4
_RPACK = _ND // _ED


def _scatter_body(ne_hbm, rcv_hbm, zeros_hbm, out_hbm,
                  ridx0, ridx1, ridx2, ridx3,
                  vals0, vals1, vals2, vals3,
                  st0, st1, st2, st3, sc0, sc1, sc2, sc3, agg_sh):
    cid = lax.axis_index("c")
    sid = lax.axis_index("s")
    wid = sid * _NC + cid
    nfull = _NCHUNK // _NW
    extra = _NCHUNK - nfull * _NW
    nch = nfull + jnp.where(wid < extra, 1, 0)
    nmax = nfull + (1 if extra else 0)
    ngrp = (nmax + _NSLOT - 1) // _NSLOT

    slots = ((ridx0, vals0, st0, sc0), (ridx1, vals1, st1, sc1),
             (ridx2, vals2, st2, sc2), (ridx3, vals3, st3, sc3))

    @pl.when(sid == 0)
    def _zero():
        pltpu.sync_copy(zeros_hbm, agg_sh)

    plsc.subcore_barrier()

    def stage(i, p):
        ri, va, st, _ = slots[p]
        ch = wid + i * _NW
        pltpu.async_copy(rcv_hbm.at[ch], ri, st)
        pltpu.async_copy(ne_hbm.at[pl.ds(ch * _GCH, _GCH)], va, st)

    def drain_stage(p):
        ri, va, st, _ = slots[p]
        pltpu.make_async_copy(rcv_hbm.at[0], ri, st).wait()
        pltpu.make_async_copy(ne_hbm.at[pl.ds(0, _GCH)], va, st).wait()

    def drain_scatter(p):
        ri, va, _, sc = slots[p]
        pltpu.make_async_copy(va, agg_sh.at[ri], sc).wait()

    for p in range(_NSLOT):
        @pl.when(p < nch)
        def _(p=p):
            stage(p, p)

    def group(g, carry):
        for p in range(_NSLOT):
            i = _NSLOT * g + p

            @pl.when(i < nch)
            def _(i=i, p=p):
                ri, va, _, sc = slots[p]
                drain_stage(p)
                pltpu.async_copy(va, agg_sh.at[ri], sc, add=True)

        for p in range(_NSLOT):
            j = _NSLOT * (g + 1) + p

            @pl.when(j < nch)
            def _(j=j, p=p):
                drain_scatter(p)
                stage(j, p)

        return carry

    lax.fori_loop(0, ngrp, group, 0)
    for p in range(_NSLOT):
        drain_scatter(p)
    plsc.subcore_barrier()

    @pl.when(sid == 0)
    def _writeback():
        pltpu.sync_copy(agg_sh, out_hbm.at[cid])


def _scatter_add(nep, rcv2d):
    mesh = plsc.VectorSubcoreMesh(core_axis_name="c", subcore_axis_name="s")
    zeros = jnp.zeros((_NP, _ED), jnp.float32)
    fn = functools.partial(
        pl.kernel,
        mesh=mesh,
        out_type=jax.ShapeDtypeStruct((_NC, _NP, _ED), jnp.float32),
        compiler_params=pltpu.CompilerParams(use_tc_tiling_on_sc=False),
        scratch_types=[
            pltpu.VMEM((_GCH,), jnp.int32),
            pltpu.VMEM((_GCH,), jnp.int32),
            pltpu.VMEM((_GCH,), jnp.int32),
            pltpu.VMEM((_GCH,), jnp.int32),
            pltpu.VMEM((_GCH, _ED), jnp.float32),
            pltpu.VMEM((_GCH, _ED), jnp.float32),
            pltpu.VMEM((_GCH, _ED), jnp.float32),
            pltpu.VMEM((_GCH, _ED), jnp.float32),
            pltpu.SemaphoreType.DMA,
            pltpu.SemaphoreType.DMA,
            pltpu.SemaphoreType.DMA,
            pltpu.SemaphoreType.DMA,
            pltpu.SemaphoreType.DMA,
            pltpu.SemaphoreType.DMA,
            pltpu.SemaphoreType.DMA,
            pltpu.SemaphoreType.DMA,
            pltpu.VMEM_SHARED((_NP, _ED), jnp.float32),
        ],
    )(_scatter_body)
    return fn(nep, rcv2d, zeros)


def _node_body(x_ref, agg_ref, w1a_ref, w1b_ref, b1_ref, w2_ref, b2_ref, out_ref):
    x = x_ref[...]
    a = agg_ref[0] + agg_ref[1]
    t = jnp.dot(x, w1a_ref[...], preferred_element_type=jnp.float32)
    t = t + jnp.dot(a, w1b_ref[...], preferred_element_type=jnp.float32)
    t = t + b1_ref[...]
    h = _gelu(t)
    out_ref[...] = x + jnp.dot(h, w2_ref[...], preferred_element_type=jnp.float32) + b2_ref[...]


def _node_mlp(nodes, agg2, w1a, w1b, b1, w2, b2):
    n, nd = nodes.shape
    ed = agg2.shape[2]
    h = w1a.shape[1]
    bn = 1000
    return pl.pallas_call(
        _node_body,
        grid=(n // bn,),
        in_specs=[
            pl.BlockSpec((bn, nd), lambda i: (i, 0)),
            pl.BlockSpec((_NC, bn, ed), lambda i: (0, i, 0)),
            pl.BlockSpec((nd, h), lambda i: (0, 0)),
            pl.BlockSpec((ed, h), lambda i: (0, 0)),
            pl.BlockSpec((1, h), lambda i: (0, 0)),
            pl.BlockSpec((h, nd), lambda i: (0, 0)),
            pl.BlockSpec((1, nd), lambda i: (0, 0)),
        ],
        out_specs=[pl.BlockSpec((bn, nd), lambda i: (i, 0))],
        out_shape=[jax.ShapeDtypeStruct((n, nd), jnp.float32)],
    )(nodes, agg2, w1a, w1b, b1.reshape(1, h), w2, b2.reshape(1, nd))[0]


def kernel(nodes, edges, receivers, senders,
           edge_W1, edge_b1, edge_W2, edge_b2,
           node_W1, node_b1, node_W2, node_b2):
    n, nd = nodes.shape
    e, ed = edges.shape
    assert (n, e, nd, ed) == (_N, _E, _ND, _ED)

    ew1a = edge_W1[:nd]
    ew1b = edge_W1[nd:2 * nd]
    ew1c = edge_W1[2 * nd:]
    nw1a = node_W1[:nd]
    nw1b = node_W1[nd:]

    snd2d = senders.reshape(_NCHUNK, _GCH)
    rcv2d = receivers.reshape(_NCHUNK, _GCH)

    ps, pr = _proj_tables(nodes, ew1a, ew1b)
    g = _gather_sum(ps, pr, snd2d, rcv2d)
    nep, out_edges = _edge_mlp(g, edges, ew1c, edge_b1, edge_W2, edge_b2)
    aggp = _scatter_add(nep, rcv2d)
    agg2 = aggp[:, :_N]
    out_nodes = _node_mlp(nodes, agg2, nw1a, nw1b, node_b1, node_W2, node_b2)
    return out_nodes, out_edges
